# Initial kernel scaffold; baseline (speedup 1.0000x reference)
import jax
import jax.numpy as jnp
from jax.experimental import pallas as pl


def kernel(h, edge_index, t1_W, t1_b, att_l0, att_r0, att_l1, att_r1, t2_W, t2_b):
    raise NotImplementedError("write your pallas kernel here")


# speedup vs baseline: 2.4365x; 2.4365x over previous
import functools

import jax
import jax.numpy as jnp
from jax import lax
from jax.experimental import pallas as pl
from jax.experimental.pallas import tpu as pltpu
from jax.experimental.pallas import tpu_sc as plsc

N = 10000
E = 320000
IN_DIM = 128
HID = 128
OUT = 64
EPS = 0.3

LANE = 128
NP = 10240
EP = 327680
EROWS = EP // LANE
REAL_EROWS = E // LANE
NC = 2
NS = 16
NHALF = NP // NC
NJUNK = NHALF + LANE
SCAN_ROWS = EROWS // NS
SCAN_CHUNK = 32
NPT = NHALF // NS
CMAX = SCAN_ROWS * LANE + LANE
OWN = NP // (NC * NS)
RCHUNK = 32
RCAP = RCHUNK * LANE + LANE
BATCH = 64

_F32 = jnp.float32
_I32 = jnp.int32


_TC_BLK = 1280
_TC_GRID = NP // _TC_BLK


def _tc_head_body(h_ref, wt_ref, b_ref, attl_ref, attr_ref,
                  h1_ref, al_ref, ar_ref):
    h1 = jnp.dot(h_ref[...], wt_ref[...], preferred_element_type=_F32)
    h1 = jnp.maximum(h1 + b_ref[...], 0.0)
    h1_ref[...] = h1
    al_ref[...] = jnp.sum(h1 * attl_ref[...], axis=1, keepdims=True)
    ar_ref[...] = jnp.sum(h1 * attr_ref[...], axis=1, keepdims=True)


def _tc_head(hp, w1t, b1, attl, attr):
    return pl.pallas_call(
        _tc_head_body,
        grid=(_TC_GRID,),
        in_specs=[
            pl.BlockSpec((_TC_BLK, IN_DIM), lambda i: (i, 0)),
            pl.BlockSpec((IN_DIM, HID), lambda i: (0, 0)),
            pl.BlockSpec((1, HID), lambda i: (0, 0)),
            pl.BlockSpec((1, HID), lambda i: (0, 0)),
            pl.BlockSpec((1, HID), lambda i: (0, 0)),
        ],
        out_specs=[
            pl.BlockSpec((_TC_BLK, HID), lambda i: (i, 0)),
            pl.BlockSpec((_TC_BLK, 1), lambda i: (i, 0)),
            pl.BlockSpec((_TC_BLK, 1), lambda i: (i, 0)),
        ],
        out_shape=[
            jax.ShapeDtypeStruct((NP, HID), _F32),
            jax.ShapeDtypeStruct((NP, 1), _F32),
            jax.ShapeDtypeStruct((NP, 1), _F32),
        ],
    )(hp, w1t, b1, attl, attr)


def _tc_mid_body(agg_ref, h1_ref, attl_ref, attr_ref,
                 h2_ref, al_ref, ar_ref):
    h2 = agg_ref[...] + EPS * h1_ref[...]
    h2_ref[...] = h2
    al_ref[...] = jnp.sum(h2 * attl_ref[...], axis=1, keepdims=True)
    ar_ref[...] = jnp.sum(h2 * attr_ref[...], axis=1, keepdims=True)


def _tc_mid(agg, h1p, attl, attr):
    return pl.pallas_call(
        _tc_mid_body,
        grid=(_TC_GRID,),
        in_specs=[
            pl.BlockSpec((_TC_BLK, HID), lambda i: (i, 0)),
            pl.BlockSpec((_TC_BLK, HID), lambda i: (i, 0)),
            pl.BlockSpec((1, HID), lambda i: (0, 0)),
            pl.BlockSpec((1, HID), lambda i: (0, 0)),
        ],
        out_specs=[
            pl.BlockSpec((_TC_BLK, HID), lambda i: (i, 0)),
            pl.BlockSpec((_TC_BLK, 1), lambda i: (i, 0)),
            pl.BlockSpec((_TC_BLK, 1), lambda i: (i, 0)),
        ],
        out_shape=[
            jax.ShapeDtypeStruct((NP, HID), _F32),
            jax.ShapeDtypeStruct((NP, 1), _F32),
            jax.ShapeDtypeStruct((NP, 1), _F32),
        ],
    )(agg, h1p, attl, attr)


def _tc_tail_body(agg_ref, h1_ref, w2t_ref, b2_ref, out_ref):
    h3 = agg_ref[...] + EPS * h1_ref[...]
    out_ref[...] = jnp.dot(h3, w2t_ref[...], preferred_element_type=_F32) \
        + b2_ref[...]


def _tc_tail(agg, h1p, w2t, b2):
    return pl.pallas_call(
        _tc_tail_body,
        grid=(_TC_GRID,),
        in_specs=[
            pl.BlockSpec((_TC_BLK, HID), lambda i: (i, 0)),
            pl.BlockSpec((_TC_BLK, HID), lambda i: (i, 0)),
            pl.BlockSpec((HID, OUT), lambda i: (0, 0)),
            pl.BlockSpec((1, OUT), lambda i: (0, 0)),
        ],
        out_specs=[pl.BlockSpec((_TC_BLK, OUT), lambda i: (i, 0))],
        out_shape=[jax.ShapeDtypeStruct((NP, OUT), _F32)],
    )(agg, h1p, w2t, b2)


def _rsqrt16(d):
    x = jnp.maximum(d, 1e-12)
    i = lax.bitcast_convert_type(x, _I32)
    i = 0x5F3759DF - (i >> 1)
    y = lax.bitcast_convert_type(i, _F32)
    for _ in range(3):
        y = y * (1.5 - 0.5 * x * y * y)
    return jnp.where(d > 0.0, y, jnp.zeros_like(y))


def _tanh16(x):
    e = jnp.exp(-2.0 * jnp.abs(x))
    return jnp.sign(x) * (1.0 - 2.0 * e / (1.0 + e))


def _scan_compact(c, s, dst_hbm, dd_v, cdst_v, extras=()):
    lo = c * NHALF

    def chunk(k, cnt):
        r0 = s * SCAN_ROWS + k * SCAN_CHUNK
        pltpu.sync_copy(dst_hbm.at[pl.ds(r0, SCAN_CHUNK)], dd_v)
        for (hbm, st, _) in extras:
            pltpu.sync_copy(hbm.at[pl.ds(r0, SCAN_CHUNK)], st)

        def row(r, cnt):
            valid = (r0 + r) < REAL_EROWS
            for i in range(8):
                off = i * 16
                dst16 = dd_v[r, pl.ds(off, 16)]
                rel = dst16 - lo
                m = (rel >= 0) & (rel < NHALF) & valid
                plsc.store_compressed(cdst_v.at[pl.ds(cnt, 16)], rel, mask=m)
                for (_, st, out) in extras:
                    plsc.store_compressed(out.at[pl.ds(cnt, 16)],
                                          st[r, pl.ds(off, 16)], mask=m)
                cnt = cnt + plsc.all_reduce_population_count(m)[0]
            return cnt

        return lax.fori_loop(0, SCAN_CHUNK, row, cnt)

    return lax.fori_loop(0, SCAN_ROWS // SCAN_CHUNK, chunk, 0)


def _pad_fill(cnt, rnd, cdst_v, extras=()):
    def fill(k, _):
        off = cnt + k * 16

        @pl.when(off < rnd)
        def _():
            cdst_v[pl.ds(off, 16)] = jnp.full((16,), NHALF, _I32)
            for (_, _, out) in extras:
                out[pl.ds(off, 16)] = jnp.zeros((16,), _I32)
        return 0

    lax.fori_loop(0, 8, fill, 0)


def _deg_body(dst_hbm, dinv_out,
              dd_v, cdst_v, idx2_v, ones_v, nv_v, deg_s):
    c = lax.axis_index("c")
    s = lax.axis_index("s")

    for i in range(21):
        nv_v[pl.ds(i * 16, 16)] = jnp.zeros((16,), _F32)
    pltpu.sync_copy(nv_v.at[pl.ds(0, NJUNK // NS)],
                    deg_s.at[pl.ds(s * (NJUNK // NS), NJUNK // NS)])
    for i in range(8):
        ones_v[pl.ds(i * 16, 16)] = jnp.full((16,), 1.0, _F32)
    plsc.subcore_barrier()

    cnt = _scan_compact(c, s, dst_hbm, dd_v, cdst_v)
    nb = (cnt + LANE - 1) // LANE
    _pad_fill(cnt, nb * LANE, cdst_v)

    def batch(b, _):
        for i in range(8):
            off = i * 16
            idx2_v[0, pl.ds(off, 16)] = cdst_v[pl.ds(b * LANE + off, 16)]
        pltpu.sync_copy(ones_v, deg_s.at[idx2_v.at[0]], add=True)
        return 0

    lax.fori_loop(0, nb, batch, 0)
    plsc.subcore_barrier()

    pltpu.sync_copy(deg_s.at[pl.ds(s * NPT, NPT)], nv_v.at[pl.ds(0, NPT)])

    def dinv_blk(j, _):
        off = j * 16
        nv_v[pl.ds(off, 16)] = _rsqrt16(nv_v[pl.ds(off, 16)])
        return 0

    lax.fori_loop(0, NPT // 16, dinv_blk, 0)
    pltpu.sync_copy(nv_v.at[pl.ds(0, NPT)],
                    dinv_out.at[pl.ds(c * NHALF + s * NPT, NPT)])


def _layer_body(src_hbm, dst_hbm, h_hbm, al_hbm, ar_hbm, dinv_hbm,
                agg_out,
                al_v, ar_v, dinv_v, dd_v, ds_v, cdst_v, csrc_v,
                w_v, rows_v, acc_v, sem):
    c = lax.axis_index("c")
    s = lax.axis_index("s")
    g = c * NS + s
    lo = g * OWN

    pltpu.sync_copy(al_hbm, al_v)
    pltpu.sync_copy(ar_hbm, ar_v)
    pltpu.sync_copy(dinv_hbm, dinv_v)

    def zacc(r, _):
        for i in range(8):
            acc_v[r, pl.ds(i * 16, 16)] = jnp.zeros((16,), _F32)
        return 0

    lax.fori_loop(0, OWN + 1, zacc, 0)

    def rnd(k, _):
        r0 = k * RCHUNK
        pltpu.sync_copy(dst_hbm.at[pl.ds(r0, RCHUNK)], dd_v)
        pltpu.sync_copy(src_hbm.at[pl.ds(r0, RCHUNK)], ds_v)

        def row(r, cnt):
            valid = (r0 + r) < REAL_EROWS
            for i in range(8):
                off = i * 16
                rel = dd_v[r, pl.ds(off, 16)] - lo
                m = (rel >= 0) & (rel < OWN) & valid
                plsc.store_compressed(cdst_v.at[pl.ds(cnt, 16)], rel,
                                      mask=m)
                plsc.store_compressed(csrc_v.at[pl.ds(cnt, 16)],
                                      ds_v[r, pl.ds(off, 16)], mask=m)
                cnt = cnt + plsc.all_reduce_population_count(m)[0]
            return cnt

        cnt = lax.fori_loop(0, RCHUNK, row, 0)
        nb = (cnt + BATCH - 1) // BATCH

        def fill(j, _):
            off2 = cnt + j * 16

            @pl.when(off2 < nb * BATCH)
            def _():
                cdst_v[pl.ds(off2, 16)] = jnp.full((16,), OWN, _I32)
                csrc_v[pl.ds(off2, 16)] = jnp.zeros((16,), _I32)
            return 0

        lax.fori_loop(0, BATCH // 16, fill, 0)

        def batch(b, _):
            e0 = b * BATCH
            cp = pltpu.async_copy(h_hbm.at[csrc_v.at[pl.ds(e0, BATCH)]],
                                  rows_v, sem)

            for i in range(BATCH // 16):
                off = i * 16
                isrc = csrc_v[pl.ds(e0 + off, 16)]
                idst = jnp.minimum(cdst_v[pl.ds(e0 + off, 16)] + lo,
                                   NP - 1)
                als = plsc.load_gather(al_v, [isrc])
                ard = plsc.load_gather(ar_v, [idst])
                dis = plsc.load_gather(dinv_v, [isrc])
                did = plsc.load_gather(dinv_v, [idst])
                w_v[pl.ds(off, 16)] = _tanh16(als + ard) * dis * did

            cp.wait()

            def scale(j, _):
                w16 = w_v[pl.ds(j * 16, 16)]
                d16 = cdst_v[pl.ds(e0 + j * 16, 16)]
                for l in range(16):
                    e = j * 16 + l
                    we = w16[l]
                    de = d16[l]
                    for kk in range(8):
                        o2 = kk * 16
                        acc_v[de, pl.ds(o2, 16)] = (
                            acc_v[de, pl.ds(o2, 16)]
                            + rows_v[e, pl.ds(o2, 16)] * we)
                return 0

            lax.fori_loop(0, BATCH // 16, scale, 0)
            return 0

        lax.fori_loop(0, nb, batch, 0)
        return 0

    lax.fori_loop(0, EROWS // RCHUNK, rnd, 0)

    pltpu.sync_copy(acc_v.at[pl.ds(0, OWN)], agg_out.at[pl.ds(lo, OWN)])


@functools.lru_cache(maxsize=None)
def _make_sc_deg():
    mesh = plsc.VectorSubcoreMesh(core_axis_name="c", subcore_axis_name="s",
                                  num_cores=NC, num_subcores=NS)
    scratch = [
        pltpu.VMEM((SCAN_CHUNK, LANE), _I32),
        pltpu.VMEM((CMAX,), _I32),
        pltpu.VMEM((1, LANE), _I32),
        pltpu.VMEM((LANE,), _F32),
        pltpu.VMEM((336,), _F32),
        pltpu.VMEM_SHARED((NJUNK,), _F32),
    ]
    return pl.kernel(_deg_body,
                     out_type=[jax.ShapeDtypeStruct((NP,), _F32)],
                     mesh=mesh, scratch_types=scratch,
                     compiler_params=pltpu.CompilerParams(
                         needs_layout_passes=False))


@functools.lru_cache(maxsize=None)
def _make_sc_layer():
    mesh = plsc.VectorSubcoreMesh(core_axis_name="c", subcore_axis_name="s",
                                  num_cores=NC, num_subcores=NS)
    scratch = [
        pltpu.VMEM((NP,), _F32),
        pltpu.VMEM((NP,), _F32),
        pltpu.VMEM((NP,), _F32),
        pltpu.VMEM((RCHUNK, LANE), _I32),
        pltpu.VMEM((RCHUNK, LANE), _I32),
        pltpu.VMEM((RCAP,), _I32),
        pltpu.VMEM((RCAP,), _I32),
        pltpu.VMEM((BATCH,), _F32),
        pltpu.VMEM((BATCH, HID), _F32),
        pltpu.VMEM((OWN + 1, HID), _F32),
        pltpu.SemaphoreType.DMA,
    ]
    return pl.kernel(_layer_body,
                     out_type=[jax.ShapeDtypeStruct((NP, HID), _F32)],
                     mesh=mesh, scratch_types=scratch,
                     compiler_params=pltpu.CompilerParams(
                         needs_layout_passes=False))


def kernel(h, edge_index, t1_W, t1_b, att_l0, att_r0, att_l1, att_r1,
           t2_W, t2_b):
    hp = jnp.pad(h, ((0, NP - N), (0, 0)))
    srcp = jnp.pad(edge_index[0], (0, EP - E)).reshape(EROWS, LANE)
    dstp = jnp.pad(edge_index[1], (0, EP - E)).reshape(EROWS, LANE)

    dinv, = _make_sc_deg()(dstp)
    h1p, al0, ar0 = _tc_head(hp, t1_W.T, t1_b[None], att_l0[None],
                             att_r0[None])
    sc_layer = _make_sc_layer()
    agg1, = sc_layer(srcp, dstp, h1p, al0.reshape(NP), ar0.reshape(NP),
                     dinv)
    h2p, al1, ar1 = _tc_mid(agg1, h1p, att_l1[None], att_r1[None])
    agg2, = sc_layer(srcp, dstp, h2p, al1.reshape(NP), ar1.reshape(NP),
                     dinv)
    outp, = _tc_tail(agg2, h1p, t2_W.T, t2_b[None])
    return outp[:N]
